# baseline (speedup 1.0000x reference)
import functools

import jax
import jax.numpy as jnp
from jax import lax
from jax.experimental import pallas as pl
from jax.experimental.pallas import tpu as pltpu
from jax.experimental.pallas import tpu_sc as plsc

B = 8
N = 20000
FIGSIZE = 640.0
CONF_THRESH = 0.2
NEG = -1e30

NW = 32
ROWS = 624
BUF_ROWS = 656
GROUPS = BUF_ROWS // 16

_mesh = plsc.VectorSubcoreMesh(core_axis_name="c", subcore_axis_name="s")


@functools.partial(
    pl.kernel,
    mesh=_mesh,
    out_type=jax.ShapeDtypeStruct((NW, 16), jnp.float32),
    compiler_params=pltpu.CompilerParams(needs_layout_passes=False,
                                         use_tc_tiling_on_sc=False),
    scratch_types=[
        pltpu.VMEM((BUF_ROWS, 7), jnp.float32),
        pltpu.VMEM((BUF_ROWS, 7), jnp.float32),
        pltpu.VMEM((B * 6 * 16,), jnp.float32),
        pltpu.VMEM((16,), jnp.float32),
        pltpu.SemaphoreType.DMA,
        pltpu.SemaphoreType.DMA,
    ],
)
def _sc_partial_max(boxes_hbm, params_hbm, out_hbm, buf0, buf1, par_v,
                    res_v, sem0, sem1):
    wid = lax.axis_index("c") * 16 + lax.axis_index("s")
    row0 = wid * ROWS

    pltpu.sync_copy(params_hbm, par_v)

    lane = lax.iota(jnp.int32, 16)

    res = jnp.full((16,), NEG, jnp.float32)
    handles = {0: pltpu.async_copy(
        boxes_hbm.at[0, pl.ds(row0, BUF_ROWS)], buf0, sem0)}
    for b in range(B):
        cur = buf0 if b % 2 == 0 else buf1
        if b + 1 < B:
            nbuf = buf1 if b % 2 == 0 else buf0
            nsem = sem1 if b % 2 == 0 else sem0
            handles[b + 1] = pltpu.async_copy(
                boxes_hbm.at[b + 1, pl.ds(row0, BUF_ROWS)], nbuf, nsem)
        handles[b].wait()

        gx1 = par_v[pl.ds((b * 6 + 0) * 16, 16)]
        gy1 = par_v[pl.ds((b * 6 + 1) * 16, 16)]
        gx2 = par_v[pl.ds((b * 6 + 2) * 16, 16)]
        gy2 = par_v[pl.ds((b * 6 + 3) * 16, 16)]
        area2 = par_v[pl.ds((b * 6 + 4) * 16, 16)]
        thr = par_v[pl.ds((b * 6 + 5) * 16, 16)]

        def body(g, macc):
            ridx = lane + g * 16
            c0 = jnp.zeros((16,), jnp.int32)
            cx = plsc.load_gather(cur, [ridx, c0])
            cy = plsc.load_gather(cur, [ridx, c0 + 1])
            bw = plsc.load_gather(cur, [ridx, c0 + 2])
            bh = plsc.load_gather(cur, [ridx, c0 + 3])
            conf = plsc.load_gather(cur, [ridx, c0 + 4])
            cls_f = plsc.load_gather(cur, [ridx, c0 + 6])
            x1 = (cx - bw / 2.0) * FIGSIZE
            y1 = (cy - bh / 2.0) * FIGSIZE
            x2 = (cx + bw / 2.0) * FIGSIZE
            y2 = (cy + bh / 2.0) * FIGSIZE
            ix1 = jnp.maximum(x1, gx1)
            iy1 = jnp.maximum(y1, gy1)
            ix2 = jnp.minimum(x2, gx2)
            iy2 = jnp.minimum(y2, gy2)
            inter = jnp.maximum(ix2 - ix1, 0.0) * jnp.maximum(iy2 - iy1, 0.0)
            area1 = (x2 - x1) * (y2 - y1)
            union = area1 + area2 - inter
            valid = ((conf > CONF_THRESH) & (inter >= thr * union)
                     & (cls_f.astype(jnp.int32) == 0))
            return jnp.maximum(macc, jnp.where(valid, conf, NEG))

        macc = lax.fori_loop(
            0, GROUPS, body, jnp.full((16,), NEG, jnp.float32))
        res = jnp.where(lane == b, jnp.max(macc), res)

    res_v[...] = res
    pltpu.sync_copy(res_v, out_hbm.at[wid])


def kernel(boxes, gt, iou_thresh):
    gx1, gy1, gx2, gy2 = gt[:, 0], gt[:, 1], gt[:, 2], gt[:, 3]
    area2 = (gx2 - gx1) * (gy2 - gy1)
    thr = jnp.broadcast_to(jnp.asarray(iou_thresh, jnp.float32), (B,))
    params = jnp.stack([gx1, gy1, gx2, gy2, area2, thr], axis=1)
    params = jnp.repeat(params[:, :, None], 16, axis=2).reshape(-1)
    partials = _sc_partial_max(boxes, params)
    mx = jnp.max(partials, axis=0)[:B]
    chosen = jnp.where(mx > NEG, mx, 0.0)
    return jnp.mean(chosen), chosen


# speedup vs baseline: 5.1514x; 5.1514x over previous
import functools

import jax
import jax.numpy as jnp
from jax import lax
from jax.experimental import pallas as pl
from jax.experimental.pallas import tpu as pltpu
from jax.experimental.pallas import tpu_sc as plsc

B = 8
N = 20000
FIGSIZE = 640.0
CONF_THRESH = 0.2
NEG = -1e30

NW = 32
UNITS = 157
ROUNDS = 5

_mesh = plsc.VectorSubcoreMesh(core_axis_name="c", subcore_axis_name="s")


@functools.partial(
    pl.kernel,
    mesh=_mesh,
    out_type=jax.ShapeDtypeStruct((NW, 16), jnp.float32),
    compiler_params=pltpu.CompilerParams(needs_layout_passes=False),
    scratch_types=[
        pltpu.VMEM((7, B, 128), jnp.float32),
        pltpu.VMEM((7, B, 128), jnp.float32),
        pltpu.VMEM((B * 6 * 16,), jnp.float32),
        pltpu.VMEM((16,), jnp.float32),
        pltpu.SemaphoreType.DMA,
        pltpu.SemaphoreType.DMA,
    ],
)
def _sc_partial_max(boxes_hbm, params_hbm, out_hbm, buf0, buf1, par_v,
                    res_v, sem0, sem1):
    wid = lax.axis_index("c") * 16 + lax.axis_index("s")

    pltpu.sync_copy(params_hbm, par_v)

    lane = lax.iota(jnp.int32, 16)

    def unit_start(i):
        return pl.multiple_of(jnp.minimum(NW * i + wid, UNITS - 1) * 128, 128)

    maccs = [jnp.full((16,), NEG, jnp.float32) for _ in range(B)]
    handles = {0: pltpu.async_copy(
        boxes_hbm.at[:, :, pl.ds(unit_start(0), 128)], buf0, sem0)}
    for i in range(ROUNDS):
        cur = buf0 if i % 2 == 0 else buf1
        if i + 1 < ROUNDS:
            nbuf = buf1 if i % 2 == 0 else buf0
            nsem = sem1 if i % 2 == 0 else sem0
            handles[i + 1] = pltpu.async_copy(
                boxes_hbm.at[:, :, pl.ds(unit_start(i + 1), 128)], nbuf, nsem)
        handles[i].wait()
        colbase = lane + unit_start(i)

        for b in range(B):
            gx1 = par_v[pl.ds((b * 6 + 0) * 16, 16)]
            gy1 = par_v[pl.ds((b * 6 + 1) * 16, 16)]
            gx2 = par_v[pl.ds((b * 6 + 2) * 16, 16)]
            gy2 = par_v[pl.ds((b * 6 + 3) * 16, 16)]
            area2 = par_v[pl.ds((b * 6 + 4) * 16, 16)]
            thr = par_v[pl.ds((b * 6 + 5) * 16, 16)]

            def body(k, macc):
                s = k * 16
                cx = cur[0, b, pl.ds(s, 16)]
                cy = cur[1, b, pl.ds(s, 16)]
                bw = cur[2, b, pl.ds(s, 16)]
                bh = cur[3, b, pl.ds(s, 16)]
                conf = cur[4, b, pl.ds(s, 16)]
                cls_f = cur[6, b, pl.ds(s, 16)]
                x1 = (cx - bw / 2.0) * FIGSIZE
                y1 = (cy - bh / 2.0) * FIGSIZE
                x2 = (cx + bw / 2.0) * FIGSIZE
                y2 = (cy + bh / 2.0) * FIGSIZE
                ix1 = jnp.maximum(x1, gx1)
                iy1 = jnp.maximum(y1, gy1)
                ix2 = jnp.minimum(x2, gx2)
                iy2 = jnp.minimum(y2, gy2)
                inter = (jnp.maximum(ix2 - ix1, 0.0)
                         * jnp.maximum(iy2 - iy1, 0.0))
                area1 = (x2 - x1) * (y2 - y1)
                union = area1 + area2 - inter
                valid = ((conf > CONF_THRESH) & (inter >= thr * union)
                         & (cls_f.astype(jnp.int32) == 0)
                         & (colbase + s < N))
                return jnp.maximum(macc, jnp.where(valid, conf, NEG))

            maccs[b] = lax.fori_loop(0, 8, body, maccs[b])

    res = jnp.full((16,), NEG, jnp.float32)
    for b in range(B):
        res = jnp.where(lane == b, jnp.max(maccs[b]), res)
    res_v[...] = res
    pltpu.sync_copy(res_v, out_hbm.at[wid])


def kernel(boxes, gt, iou_thresh):
    boxes_t = jnp.transpose(boxes, (2, 0, 1))
    gx1, gy1, gx2, gy2 = gt[:, 0], gt[:, 1], gt[:, 2], gt[:, 3]
    area2 = (gx2 - gx1) * (gy2 - gy1)
    thr = jnp.broadcast_to(jnp.asarray(iou_thresh, jnp.float32), (B,))
    params = jnp.stack([gx1, gy1, gx2, gy2, area2, thr], axis=1)
    params = jnp.repeat(params[:, :, None], 16, axis=2).reshape(-1)
    partials = _sc_partial_max(boxes_t, params)
    mx = jnp.max(partials, axis=0)[:B]
    chosen = jnp.where(mx > NEG, mx, 0.0)
    return jnp.mean(chosen), chosen


# speedup vs baseline: 5.1955x; 1.0086x over previous
import functools

import jax
import jax.numpy as jnp
from jax import lax
from jax.experimental import pallas as pl
from jax.experimental.pallas import tpu as pltpu
from jax.experimental.pallas import tpu_sc as plsc

B = 8
N = 20000
FIGSIZE = 640.0
CONF_THRESH = 0.2
NEG = -1e30

NW = 32
UNITS = 157
ROUNDS = 5

_mesh = plsc.VectorSubcoreMesh(core_axis_name="c", subcore_axis_name="s")


@functools.partial(
    pl.kernel,
    mesh=_mesh,
    out_type=jax.ShapeDtypeStruct((NW, 16), jnp.float32),
    compiler_params=pltpu.CompilerParams(needs_layout_passes=False,
                                         skip_device_barrier=True),
    scratch_types=[
        pltpu.VMEM((7, B, 128), jnp.float32),
        pltpu.VMEM((7, B, 128), jnp.float32),
        pltpu.VMEM((B * 6 * 16,), jnp.float32),
        pltpu.VMEM((16,), jnp.float32),
        pltpu.SemaphoreType.DMA,
        pltpu.SemaphoreType.DMA,
    ],
)
def _sc_partial_max(boxes_hbm, params_hbm, out_hbm, buf0, buf1, par_v,
                    res_v, sem0, sem1):
    wid = lax.axis_index("c") * 16 + lax.axis_index("s")

    pltpu.sync_copy(params_hbm, par_v)

    lane = lax.iota(jnp.int32, 16)

    def unit_start(i):
        return pl.multiple_of(jnp.minimum(NW * i + wid, UNITS - 1) * 128, 128)

    maccs = [jnp.full((16,), NEG, jnp.float32) for _ in range(B)]
    handles = {0: pltpu.async_copy(
        boxes_hbm.at[:, :, pl.ds(unit_start(0), 128)], buf0, sem0)}
    for i in range(ROUNDS):
        cur = buf0 if i % 2 == 0 else buf1
        if i + 1 < ROUNDS:
            nbuf = buf1 if i % 2 == 0 else buf0
            nsem = sem1 if i % 2 == 0 else sem0
            handles[i + 1] = pltpu.async_copy(
                boxes_hbm.at[:, :, pl.ds(unit_start(i + 1), 128)], nbuf, nsem)
        handles[i].wait()
        colbase = lane + unit_start(i)

        for b in range(B):
            gx1 = par_v[pl.ds((b * 6 + 0) * 16, 16)]
            gy1 = par_v[pl.ds((b * 6 + 1) * 16, 16)]
            gx2 = par_v[pl.ds((b * 6 + 2) * 16, 16)]
            gy2 = par_v[pl.ds((b * 6 + 3) * 16, 16)]
            area2 = par_v[pl.ds((b * 6 + 4) * 16, 16)]
            thr = par_v[pl.ds((b * 6 + 5) * 16, 16)]

            def body(k, macc):
                s = k * 16
                cx = cur[0, b, pl.ds(s, 16)]
                cy = cur[1, b, pl.ds(s, 16)]
                bw = cur[2, b, pl.ds(s, 16)]
                bh = cur[3, b, pl.ds(s, 16)]
                conf = cur[4, b, pl.ds(s, 16)]
                cls_f = cur[6, b, pl.ds(s, 16)]
                x1 = (cx - bw / 2.0) * FIGSIZE
                y1 = (cy - bh / 2.0) * FIGSIZE
                x2 = (cx + bw / 2.0) * FIGSIZE
                y2 = (cy + bh / 2.0) * FIGSIZE
                ix1 = jnp.maximum(x1, gx1)
                iy1 = jnp.maximum(y1, gy1)
                ix2 = jnp.minimum(x2, gx2)
                iy2 = jnp.minimum(y2, gy2)
                inter = (jnp.maximum(ix2 - ix1, 0.0)
                         * jnp.maximum(iy2 - iy1, 0.0))
                area1 = (x2 - x1) * (y2 - y1)
                union = area1 + area2 - inter
                valid = ((conf > CONF_THRESH) & (inter >= thr * union)
                         & (cls_f.astype(jnp.int32) == 0)
                         & (colbase + s < N))
                return jnp.maximum(macc, jnp.where(valid, conf, NEG))

            maccs[b] = lax.fori_loop(0, 8, body, maccs[b])

    res = jnp.full((16,), NEG, jnp.float32)
    for b in range(B):
        res = jnp.where(lane == b, jnp.max(maccs[b]), res)
    res_v[...] = res
    pltpu.sync_copy(res_v, out_hbm.at[wid])


def kernel(boxes, gt, iou_thresh):
    boxes_t = jnp.transpose(boxes, (2, 0, 1))
    gx1, gy1, gx2, gy2 = gt[:, 0], gt[:, 1], gt[:, 2], gt[:, 3]
    area2 = (gx2 - gx1) * (gy2 - gy1)
    thr = jnp.broadcast_to(jnp.asarray(iou_thresh, jnp.float32), (B,))
    params = jnp.stack([gx1, gy1, gx2, gy2, area2, thr], axis=1)
    params = jnp.repeat(params[:, :, None], 16, axis=2).reshape(-1)
    partials = _sc_partial_max(boxes_t, params)
    mx = jnp.max(partials, axis=0)[:B]
    chosen = jnp.where(mx > NEG, mx, 0.0)
    return jnp.mean(chosen), chosen


# speedup vs baseline: 5.5863x; 1.0752x over previous
import functools

import jax
import jax.numpy as jnp
from jax import lax
from jax.experimental import pallas as pl
from jax.experimental.pallas import tpu as pltpu
from jax.experimental.pallas import tpu_sc as plsc

B = 8
N = 20000
FIGSIZE = 640.0
CONF_THRESH = 0.2
NEG = -1e30

NW = 32
UNITS = 157
ROUNDS = 5

_mesh = plsc.VectorSubcoreMesh(core_axis_name="c", subcore_axis_name="s")


@functools.partial(
    pl.kernel,
    mesh=_mesh,
    out_type=jax.ShapeDtypeStruct((NW, 16), jnp.float32),
    compiler_params=pltpu.CompilerParams(needs_layout_passes=False,
                                         skip_device_barrier=True),
    scratch_types=[
        pltpu.VMEM((7, B, 128), jnp.float32),
        pltpu.VMEM((7, B, 128), jnp.float32),
        pltpu.VMEM((B * 16,), jnp.float32),
        pltpu.VMEM((16,), jnp.float32),
        pltpu.VMEM((4 * B,), jnp.float32),
        pltpu.VMEM((16,), jnp.float32),
        pltpu.VMEM((B * 5 * 16,), jnp.float32),
        pltpu.SemaphoreType.DMA,
        pltpu.SemaphoreType.DMA,
    ],
)
def _sc_partial_max(boxes_hbm, gt_hbm, thr_hbm, out_hbm, buf0, buf1, mres,
                    res_v, gt_s, thr_s, parm, sem0, sem1):
    wid = lax.axis_index("c") * 16 + lax.axis_index("s")

    pltpu.sync_copy(gt_hbm, gt_s)
    pltpu.sync_copy(thr_hbm, thr_s)

    lane = lax.iota(jnp.int32, 16)
    neg_vec = jnp.full((16,), NEG, jnp.float32)

    thrv = thr_s[...]
    g_lo = gt_s[pl.ds(0, 16)]
    g_hi = gt_s[pl.ds(16, 16)]
    for b in range(B):
        gx1 = g_lo[b]
        gy1 = g_lo[B + b]
        gx2 = g_hi[b]
        gy2 = g_hi[B + b]
        parm[pl.ds((b * 5 + 0) * 16, 16)] = jnp.full((16,), gx1, jnp.float32)
        parm[pl.ds((b * 5 + 1) * 16, 16)] = jnp.full((16,), gy1, jnp.float32)
        parm[pl.ds((b * 5 + 2) * 16, 16)] = jnp.full((16,), gx2, jnp.float32)
        parm[pl.ds((b * 5 + 3) * 16, 16)] = jnp.full((16,), gy2, jnp.float32)
        parm[pl.ds((b * 5 + 4) * 16, 16)] = jnp.full(
            (16,), (gx2 - gx1) * (gy2 - gy1), jnp.float32)

    def unit_start(i):
        return pl.multiple_of(jnp.minimum(NW * i + wid, UNITS - 1) * 128, 128)

    for b in range(B):
        mres[pl.ds(b * 16, 16)] = neg_vec

    handles = {0: pltpu.async_copy(
        boxes_hbm.at[:, :, pl.ds(unit_start(0), 128)], buf0, sem0)}
    for i in range(ROUNDS):
        cur = buf0 if i % 2 == 0 else buf1
        if i + 1 < ROUNDS:
            nbuf = buf1 if i % 2 == 0 else buf0
            nsem = sem1 if i % 2 == 0 else sem0
            handles[i + 1] = pltpu.async_copy(
                boxes_hbm.at[:, :, pl.ds(unit_start(i + 1), 128)], nbuf, nsem)
        handles[i].wait()
        colbase = lane + unit_start(i)

        def bbody(b, _):
            p0 = b * 80
            gx1 = parm[pl.ds(p0, 16)]
            gy1 = parm[pl.ds(p0 + 16, 16)]
            gx2 = parm[pl.ds(p0 + 32, 16)]
            gy2 = parm[pl.ds(p0 + 48, 16)]
            area2 = parm[pl.ds(p0 + 64, 16)]
            thr = thrv

            def kbody(k, macc):
                s = k * 16
                cx = cur[0, b, pl.ds(s, 16)]
                cy = cur[1, b, pl.ds(s, 16)]
                bw = cur[2, b, pl.ds(s, 16)]
                bh = cur[3, b, pl.ds(s, 16)]
                conf = cur[4, b, pl.ds(s, 16)]
                cls_f = cur[6, b, pl.ds(s, 16)]
                x1 = (cx - bw / 2.0) * FIGSIZE
                y1 = (cy - bh / 2.0) * FIGSIZE
                x2 = (cx + bw / 2.0) * FIGSIZE
                y2 = (cy + bh / 2.0) * FIGSIZE
                ix1 = jnp.maximum(x1, gx1)
                iy1 = jnp.maximum(y1, gy1)
                ix2 = jnp.minimum(x2, gx2)
                iy2 = jnp.minimum(y2, gy2)
                inter = (jnp.maximum(ix2 - ix1, 0.0)
                         * jnp.maximum(iy2 - iy1, 0.0))
                area1 = (x2 - x1) * (y2 - y1)
                union = area1 + area2 - inter
                valid = ((conf > CONF_THRESH) & (inter >= thr * union)
                         & (cls_f.astype(jnp.int32) == 0)
                         & (colbase + s < N))
                return jnp.maximum(macc, jnp.where(valid, conf, NEG))

            mres[pl.ds(b * 16, 16)] = lax.fori_loop(
                0, 8, kbody, mres[pl.ds(b * 16, 16)])
            return 0

        lax.fori_loop(0, B, bbody, 0)

    res = neg_vec
    for b in range(B):
        res = jnp.where(lane == b, jnp.max(mres[pl.ds(b * 16, 16)]), res)
    res_v[...] = res
    pltpu.sync_copy(res_v, out_hbm.at[wid])


def kernel(boxes, gt, iou_thresh):
    boxes_t = jnp.transpose(boxes, (2, 0, 1))
    gt_t = gt.T.reshape(4 * B)
    thr1 = jnp.broadcast_to(jnp.asarray(iou_thresh, jnp.float32), (16,))
    partials = _sc_partial_max(boxes_t, gt_t, thr1)
    mx = jnp.max(partials, axis=0)[:B]
    chosen = jnp.where(mx > NEG, mx, 0.0)
    return jnp.mean(chosen), chosen
